# baseline (speedup 1.0000x reference)
import functools

import jax
import jax.numpy as jnp
from jax import lax
from jax.experimental import pallas as pl
from jax.experimental.pallas import tpu as pltpu

K = 8
NUM_CLASSES = 100
INPUT_DIM = NUM_CLASSES + 1
C = K * INPUT_DIM
LM = 42
LP = 48
NSPLIT = 4


def _sub_block_sums(x, lw8, tw8, validf, e1, e2):
    NR = x.shape[0]

    ex = jnp.exp(x)
    s8 = lax.dot_general(
        ex, e1, (((1,), (1,)), ((), ())), preferred_element_type=jnp.float32
    )
    lse_sum = jnp.sum(jnp.log(s8) * validf)

    ki = jax.lax.broadcasted_iota(jnp.int32, (NR, K), 1)
    tlane = jnp.where(validf > 0, ki * INPUT_DIM + lw8, -1).astype(jnp.float32)
    t_exp = lax.dot_general(
        tlane, e2, (((1,), (0,)), ((), ())), preferred_element_type=jnp.float32
    )
    lanei = jax.lax.broadcasted_iota(jnp.int32, (NR, C), 1)
    tgt_sum = jnp.sum(jnp.where(lanei == t_exp.astype(jnp.int32), x, 0.0))

    tw_exp = lax.dot_general(
        tw8, e2, (((1,), (0,)), ((), ())), preferred_element_type=jnp.float32
    )
    iseg = lanei // INPUT_DIM
    is_time = (lanei - iseg * INPUT_DIM) == NUM_CLASSES
    d = x - tw_exp
    mse_sum = jnp.sum(jnp.where(is_time, d * d, 0.0) * validf)

    return lse_sum - tgt_sum, mse_sum


def _loss_body(len_ref, *refs):
    pred_refs = refs[:NSPLIT]
    lw_ref, tw_ref, out_ref = refs[NSPLIT:]
    i = pl.program_id(0)

    @pl.when(i == 0)
    def _init():
        out_ref[...] = jnp.zeros_like(out_ref)

    BS = pred_refs[0].shape[0]
    NR = BS * LP

    jlane = jax.lax.broadcasted_iota(jnp.int32, (K, C), 1)
    trow = jax.lax.broadcasted_iota(jnp.int32, (K, C), 0)
    jseg = jlane // INPUT_DIM
    jcls = jlane - jseg * INPUT_DIM
    e2 = (jseg == trow).astype(jnp.float32)
    e1 = e2 * (jcls < NUM_CLASSES).astype(jnp.float32)

    rowi = jax.lax.broadcasted_iota(jnp.int32, (NR, 1), 0)
    posr = rowi - (rowi // LP) * LP

    ce_total = jnp.float32(0.0)
    mse_total = jnp.float32(0.0)
    cnt_total = jnp.float32(0.0)
    for j in range(NSPLIT):
        x = pred_refs[j][...].reshape(NR, C)
        lw8 = lw_ref[j * BS : (j + 1) * BS].reshape(NR, K)
        tw8 = tw_ref[j * BS : (j + 1) * BS].reshape(NR, K)
        lenr = len_ref[j * BS : (j + 1) * BS].reshape(NR, 1)
        validf = (posr < lenr).astype(jnp.float32)
        ce_j, mse_j = _sub_block_sums(x, lw8, tw8, validf, e1, e2)
        ce_total += ce_j
        mse_total += mse_j
        cnt_total += jnp.sum(validf)

    olane = jax.lax.broadcasted_iota(jnp.int32, (1, 128), 1)
    upd = (
        jnp.where(olane == 0, ce_total, 0.0)
        + jnp.where(olane == 1, mse_total, 0.0)
        + jnp.where(olane == 2, cnt_total, 0.0)
    )
    out_ref[...] += upd


@functools.partial(jax.jit, static_argnames=("bb",))
def _next_k_loss(predictions, labels, timestamps, seq_lens, bb=32):
    B, L, _ = predictions.shape
    bs = bb // NSPLIT
    lengths = jnp.clip(seq_lens - K, 0, LM).astype(jnp.int32)

    labp = jnp.concatenate(
        [labels.astype(jnp.int32), jnp.zeros((B, LP + K - L), jnp.int32)], axis=1
    )
    tsp = jnp.concatenate(
        [timestamps, jnp.zeros((B, LP + K - L), jnp.float32)], axis=1
    )
    lw = jnp.stack([labp[:, 1 + k : 1 + k + LP] for k in range(K)], axis=-1)
    tw = jnp.stack([tsp[:, 1 + k : 1 + k + LP] for k in range(K)], axis=-1)
    lenexp = jnp.broadcast_to(lengths[:, None, None], (B, LP, 1))

    grid = (B // bb,)
    pred_specs = [
        pl.BlockSpec((bs, LP, C), functools.partial(lambda i, jj: (NSPLIT * i + jj, 0, 0), jj=j))
        for j in range(NSPLIT)
    ]
    out = pl.pallas_call(
        _loss_body,
        grid=grid,
        in_specs=[
            pl.BlockSpec((bb, LP, 1), lambda i: (i, 0, 0)),
            *pred_specs,
            pl.BlockSpec((bb, LP, K), lambda i: (i, 0, 0)),
            pl.BlockSpec((bb, LP, K), lambda i: (i, 0, 0)),
        ],
        out_specs=pl.BlockSpec((1, 128), lambda i: (0, 0)),
        out_shape=jax.ShapeDtypeStruct((1, 128), jnp.float32),
    )(lenexp, predictions, predictions, predictions, predictions, lw, tw)

    ce_sum = out[0, 0]
    mse_sum = out[0, 1]
    denom = jnp.maximum(out[0, 2] * K, 1.0)
    return jnp.stack([ce_sum / denom, mse_sum / denom])


def kernel(predictions, labels, timestamps, seq_lens):
    return _next_k_loss(predictions, labels, timestamps, seq_lens)


# speedup vs baseline: 2.2643x; 2.2643x over previous
import functools

import jax
import jax.numpy as jnp
from jax import lax
from jax.experimental import pallas as pl
from jax.experimental.pallas import tpu as pltpu

K = 8
NUM_CLASSES = 100
INPUT_DIM = NUM_CLASSES + 1
C = K * INPUT_DIM
LM = 42
LP = 48
NSPLIT = 4


def _sub_block_sums(x, lw8, tw8, validf, e1, e2):
    NR = x.shape[0]

    ex = jnp.exp(x)
    s8 = lax.dot_general(
        ex, e1, (((1,), (1,)), ((), ())), preferred_element_type=jnp.float32
    )
    lse_sum = jnp.sum(jnp.log(s8) * validf)

    ki = jax.lax.broadcasted_iota(jnp.int32, (NR, K), 1)
    tlane = jnp.where(validf > 0, ki * INPUT_DIM + lw8, -1).astype(jnp.float32)
    t_exp = lax.dot_general(
        tlane, e2, (((1,), (0,)), ((), ())), preferred_element_type=jnp.float32
    )
    lanei = jax.lax.broadcasted_iota(jnp.int32, (NR, C), 1)
    tgt_sum = jnp.sum(jnp.where(lanei == t_exp.astype(jnp.int32), x, 0.0))

    tw_exp = lax.dot_general(
        tw8, e2, (((1,), (0,)), ((), ())), preferred_element_type=jnp.float32
    )
    iseg = lanei // INPUT_DIM
    is_time = (lanei - iseg * INPUT_DIM) == NUM_CLASSES
    d = x - tw_exp
    mse_sum = jnp.sum(jnp.where(is_time, d * d, 0.0) * validf)

    return lse_sum - tgt_sum, mse_sum


def _loss_body(len_ref, *refs):
    pred_refs = refs[:NSPLIT]
    lw_ref, tw_ref, out_ref = refs[NSPLIT:]
    i = pl.program_id(0)

    @pl.when(i == 0)
    def _init():
        out_ref[...] = jnp.zeros_like(out_ref)

    BS = pred_refs[0].shape[0]
    NR = BS * LP

    jlane = jax.lax.broadcasted_iota(jnp.int32, (K, C), 1)
    trow = jax.lax.broadcasted_iota(jnp.int32, (K, C), 0)
    jseg = jlane // INPUT_DIM
    jcls = jlane - jseg * INPUT_DIM
    e2 = (jseg == trow).astype(jnp.float32)
    e1 = e2 * (jcls < NUM_CLASSES).astype(jnp.float32)

    rowi = jax.lax.broadcasted_iota(jnp.int32, (NR, 1), 0)
    posr = rowi - (rowi // LP) * LP

    ce_total = jnp.float32(0.0)
    mse_total = jnp.float32(0.0)
    cnt_total = jnp.float32(0.0)
    for j in range(NSPLIT):
        x = pred_refs[j][...].reshape(NR, C)
        lw8 = lw_ref[j * BS : (j + 1) * BS].reshape(NR, K)
        tw8 = tw_ref[j * BS : (j + 1) * BS].reshape(NR, K)
        lenr = len_ref[j * BS : (j + 1) * BS].reshape(NR, 1)
        validf = (posr < lenr).astype(jnp.float32)
        ce_j, mse_j = _sub_block_sums(x, lw8, tw8, validf, e1, e2)
        ce_total += ce_j
        mse_total += mse_j
        cnt_total += jnp.sum(validf)

    olane = jax.lax.broadcasted_iota(jnp.int32, (1, 128), 1)
    upd = (
        jnp.where(olane == 0, ce_total, 0.0)
        + jnp.where(olane == 1, mse_total, 0.0)
        + jnp.where(olane == 2, cnt_total, 0.0)
    )
    out_ref[...] += upd


@functools.partial(jax.jit, static_argnames=("bb",))
def _next_k_loss(predictions, labels, timestamps, seq_lens, bb=32):
    B, L, _ = predictions.shape
    bs = bb // NSPLIT
    lengths = jnp.clip(seq_lens - K, 0, LM).astype(jnp.int32)

    labp = jnp.concatenate(
        [labels.astype(jnp.int32), jnp.zeros((B, LP + K - L), jnp.int32)], axis=1
    )
    tsp = jnp.concatenate(
        [timestamps, jnp.zeros((B, LP + K - L), jnp.float32)], axis=1
    )
    widx = jnp.arange(LP)[:, None] + 1 + jnp.arange(K)[None, :]
    lw = labp[:, widx]
    tw = tsp[:, widx]
    lenexp = jnp.broadcast_to(lengths[:, None, None], (B, LP, 1))

    grid = (B // bb,)
    pred_specs = [
        pl.BlockSpec((bs, LP, C), functools.partial(lambda i, jj: (NSPLIT * i + jj, 0, 0), jj=j))
        for j in range(NSPLIT)
    ]
    out = pl.pallas_call(
        _loss_body,
        grid=grid,
        in_specs=[
            pl.BlockSpec((bb, LP, 1), lambda i: (i, 0, 0)),
            *pred_specs,
            pl.BlockSpec((bb, LP, K), lambda i: (i, 0, 0)),
            pl.BlockSpec((bb, LP, K), lambda i: (i, 0, 0)),
        ],
        out_specs=pl.BlockSpec((1, 128), lambda i: (0, 0)),
        out_shape=jax.ShapeDtypeStruct((1, 128), jnp.float32),
    )(lenexp, predictions, predictions, predictions, predictions, lw, tw)

    ce_sum = out[0, 0]
    mse_sum = out[0, 1]
    denom = jnp.maximum(out[0, 2] * K, 1.0)
    return jnp.stack([ce_sum / denom, mse_sum / denom])


def kernel(predictions, labels, timestamps, seq_lens):
    return _next_k_loss(predictions, labels, timestamps, seq_lens)


# speedup vs baseline: 2.4315x; 1.0738x over previous
import functools

import jax
import jax.numpy as jnp
from jax import lax
from jax.experimental import pallas as pl
from jax.experimental.pallas import tpu as pltpu

K = 8
NUM_CLASSES = 100
INPUT_DIM = NUM_CLASSES + 1
C = K * INPUT_DIM
LM = 42
LP = 48
NSPLIT = 4


def _sub_block_sums(x, lw8, tw8, validf, e1, e2):
    NR = x.shape[0]

    ex = jnp.exp(x)
    s8 = lax.dot_general(
        ex, e1, (((1,), (1,)), ((), ())), preferred_element_type=jnp.float32
    )
    lse_sum = jnp.sum(jnp.log(s8) * validf)

    ki = jax.lax.broadcasted_iota(jnp.int32, (NR, K), 1)
    tlane = jnp.where(validf > 0, ki * INPUT_DIM + lw8, -1).astype(jnp.float32)
    t_exp = lax.dot_general(
        tlane, e2, (((1,), (0,)), ((), ())), preferred_element_type=jnp.float32
    )
    lanei = jax.lax.broadcasted_iota(jnp.int32, (NR, C), 1)
    tgt_sum = jnp.sum(jnp.where(lanei == t_exp.astype(jnp.int32), x, 0.0))

    x_time = lax.dot_general(
        x, e2 - e1, (((1,), (1,)), ((), ())), preferred_element_type=jnp.float32
    )
    d8 = x_time - tw8
    mse_sum = jnp.sum(d8 * d8 * validf)

    return lse_sum - tgt_sum, mse_sum


def _loss_body(len_ref, *refs):
    pred_refs = refs[:NSPLIT]
    lw_ref, tw_ref, out_ref = refs[NSPLIT:]
    i = pl.program_id(0)

    @pl.when(i == 0)
    def _init():
        out_ref[...] = jnp.zeros_like(out_ref)

    BS = pred_refs[0].shape[0]
    NR = BS * LP

    jlane = jax.lax.broadcasted_iota(jnp.int32, (K, C), 1)
    trow = jax.lax.broadcasted_iota(jnp.int32, (K, C), 0)
    jseg = jlane // INPUT_DIM
    jcls = jlane - jseg * INPUT_DIM
    e2 = (jseg == trow).astype(jnp.float32)
    e1 = e2 * (jcls < NUM_CLASSES).astype(jnp.float32)

    rowi = jax.lax.broadcasted_iota(jnp.int32, (NR, 1), 0)
    posr = rowi - (rowi // LP) * LP

    ce_total = jnp.float32(0.0)
    mse_total = jnp.float32(0.0)
    cnt_total = jnp.float32(0.0)
    for j in range(NSPLIT):
        x = pred_refs[j][...].reshape(NR, C)
        lw8 = lw_ref[j * BS : (j + 1) * BS].reshape(NR, K)
        tw8 = tw_ref[j * BS : (j + 1) * BS].reshape(NR, K)
        lenr = len_ref[j * BS : (j + 1) * BS].reshape(NR, 1)
        validf = (posr < lenr).astype(jnp.float32)
        ce_j, mse_j = _sub_block_sums(x, lw8, tw8, validf, e1, e2)
        ce_total += ce_j
        mse_total += mse_j
        cnt_total += jnp.sum(validf)

    olane = jax.lax.broadcasted_iota(jnp.int32, (1, 128), 1)
    upd = (
        jnp.where(olane == 0, ce_total, 0.0)
        + jnp.where(olane == 1, mse_total, 0.0)
        + jnp.where(olane == 2, cnt_total, 0.0)
    )
    out_ref[...] += upd


@functools.partial(jax.jit, static_argnames=("bb",))
def _next_k_loss(predictions, labels, timestamps, seq_lens, bb=32):
    B, L, _ = predictions.shape
    bs = bb // NSPLIT
    lengths = jnp.clip(seq_lens - K, 0, LM).astype(jnp.int32)

    labp = jnp.concatenate(
        [labels.astype(jnp.int32), jnp.zeros((B, LP + K - L), jnp.int32)], axis=1
    )
    tsp = jnp.concatenate(
        [timestamps, jnp.zeros((B, LP + K - L), jnp.float32)], axis=1
    )
    widx = jnp.arange(LP)[:, None] + 1 + jnp.arange(K)[None, :]
    lw = labp[:, widx]
    tw = tsp[:, widx]
    lenexp = jnp.broadcast_to(lengths[:, None, None], (B, LP, 1))

    grid = (B // bb,)
    pred_specs = [
        pl.BlockSpec((bs, LP, C), functools.partial(lambda i, jj: (NSPLIT * i + jj, 0, 0), jj=j))
        for j in range(NSPLIT)
    ]
    out = pl.pallas_call(
        _loss_body,
        grid=grid,
        in_specs=[
            pl.BlockSpec((bb, LP, 1), lambda i: (i, 0, 0)),
            *pred_specs,
            pl.BlockSpec((bb, LP, K), lambda i: (i, 0, 0)),
            pl.BlockSpec((bb, LP, K), lambda i: (i, 0, 0)),
        ],
        out_specs=pl.BlockSpec((1, 128), lambda i: (0, 0)),
        out_shape=jax.ShapeDtypeStruct((1, 128), jnp.float32),
    )(lenexp, predictions, predictions, predictions, predictions, lw, tw)

    ce_sum = out[0, 0]
    mse_sum = out[0, 1]
    denom = jnp.maximum(out[0, 2] * K, 1.0)
    return jnp.stack([ce_sum / denom, mse_sum / denom])


def kernel(predictions, labels, timestamps, seq_lens):
    return _next_k_loss(predictions, labels, timestamps, seq_lens)


# speedup vs baseline: 2.4654x; 1.0140x over previous
import functools

import jax
import jax.numpy as jnp
from jax import lax
from jax.experimental import pallas as pl
from jax.experimental.pallas import tpu as pltpu

K = 8
NUM_CLASSES = 100
INPUT_DIM = NUM_CLASSES + 1
C = K * INPUT_DIM
LM = 42
LP = 48
NSPLIT = 4


def _sub_block_sums(x, lw8, tw8, validf, e1, e2):
    NR = x.shape[0]

    ex = jnp.exp(x)
    s8 = lax.dot_general(
        ex, e1, (((1,), (1,)), ((), ())), preferred_element_type=jnp.float32
    )
    lse_sum = jnp.sum(jnp.log(s8) * validf)

    ki = jax.lax.broadcasted_iota(jnp.int32, (NR, K), 1)
    tlane = jnp.where(validf > 0, ki * INPUT_DIM + lw8, -1).astype(jnp.float32)
    t_exp = lax.dot_general(
        tlane, e2, (((1,), (0,)), ((), ())), preferred_element_type=jnp.float32
    )
    lanei = jax.lax.broadcasted_iota(jnp.int32, (NR, C), 1)
    tgt_sum = jnp.sum(jnp.where(lanei == t_exp.astype(jnp.int32), x, 0.0))

    x_time = lax.dot_general(
        x, e2 - e1, (((1,), (1,)), ((), ())), preferred_element_type=jnp.float32
    )
    d8 = x_time - tw8
    mse_sum = jnp.sum(d8 * d8 * validf)

    return lse_sum - tgt_sum, mse_sum


def _loss_body(len_ref, *refs):
    pred_refs = refs[:NSPLIT]
    lw_ref, tw_ref, out_ref = refs[NSPLIT:]
    i = pl.program_id(0)

    @pl.when(i == 0)
    def _init():
        out_ref[...] = jnp.zeros_like(out_ref)

    BS = pred_refs[0].shape[0]
    NR = BS * LP

    jlane = jax.lax.broadcasted_iota(jnp.int32, (K, C), 1)
    trow = jax.lax.broadcasted_iota(jnp.int32, (K, C), 0)
    jseg = jlane // INPUT_DIM
    jcls = jlane - jseg * INPUT_DIM
    e2 = (jseg == trow).astype(jnp.float32)
    e1 = e2 * (jcls < NUM_CLASSES).astype(jnp.float32)

    rowi = jax.lax.broadcasted_iota(jnp.int32, (NR, 1), 0)
    posr = rowi - (rowi // LP) * LP

    ce_total = jnp.float32(0.0)
    mse_total = jnp.float32(0.0)
    cnt_total = jnp.float32(0.0)
    for j in range(NSPLIT):
        x = pred_refs[j][...].reshape(NR, C)
        lw8 = lw_ref[j * BS : (j + 1) * BS].reshape(NR, K)
        tw8 = tw_ref[j * BS : (j + 1) * BS].reshape(NR, K)
        lenr = len_ref[j * BS : (j + 1) * BS].reshape(NR, 1)
        validf = (posr < lenr).astype(jnp.float32)
        ce_j, mse_j = _sub_block_sums(x, lw8, tw8, validf, e1, e2)
        ce_total += ce_j
        mse_total += mse_j
        cnt_total += jnp.sum(validf)

    olane = jax.lax.broadcasted_iota(jnp.int32, (1, 128), 1)
    upd = (
        jnp.where(olane == 0, ce_total, 0.0)
        + jnp.where(olane == 1, mse_total, 0.0)
        + jnp.where(olane == 2, cnt_total, 0.0)
    )
    out_ref[...] += upd


@functools.partial(jax.jit, static_argnames=("bb",))
def _next_k_loss(predictions, labels, timestamps, seq_lens, bb=64):
    B, L, _ = predictions.shape
    bs = bb // NSPLIT
    lengths = jnp.clip(seq_lens - K, 0, LM).astype(jnp.int32)

    labp = jnp.concatenate(
        [labels.astype(jnp.int32), jnp.zeros((B, LP + K - L), jnp.int32)], axis=1
    )
    tsp = jnp.concatenate(
        [timestamps, jnp.zeros((B, LP + K - L), jnp.float32)], axis=1
    )
    widx = jnp.arange(LP)[:, None] + 1 + jnp.arange(K)[None, :]
    lw = labp[:, widx]
    tw = tsp[:, widx]
    lenexp = jnp.broadcast_to(lengths[:, None, None], (B, LP, 1))

    grid = (B // bb,)
    pred_specs = [
        pl.BlockSpec((bs, LP, C), functools.partial(lambda i, jj: (NSPLIT * i + jj, 0, 0), jj=j))
        for j in range(NSPLIT)
    ]
    out = pl.pallas_call(
        _loss_body,
        grid=grid,
        in_specs=[
            pl.BlockSpec((bb, LP, 1), lambda i: (i, 0, 0)),
            *pred_specs,
            pl.BlockSpec((bb, LP, K), lambda i: (i, 0, 0)),
            pl.BlockSpec((bb, LP, K), lambda i: (i, 0, 0)),
        ],
        out_specs=pl.BlockSpec((1, 128), lambda i: (0, 0)),
        out_shape=jax.ShapeDtypeStruct((1, 128), jnp.float32),
    )(lenexp, predictions, predictions, predictions, predictions, lw, tw)

    ce_sum = out[0, 0]
    mse_sum = out[0, 1]
    denom = jnp.maximum(out[0, 2] * K, 1.0)
    return jnp.stack([ce_sum / denom, mse_sum / denom])


def kernel(predictions, labels, timestamps, seq_lens):
    return _next_k_loss(predictions, labels, timestamps, seq_lens)
